# Initial kernel scaffold; baseline (speedup 1.0000x reference)
import functools

import jax
import jax.numpy as jnp
from jax import lax
from jax.experimental import pallas as pl
from jax.experimental.pallas import tpu as pltpu
from jax.experimental.pallas import tpu_sc as plsc

DIM = 64
NC = 2
NS = 16
NW = NC * NS
IPS = 128
SUB = 5
C_ROWS = SUB * IPS


def _make_gather(batch):
    bpw = batch // NW
    nsteps = bpw // C_ROWS
    rows2_pw = bpw // IPS

    mesh = plsc.VectorSubcoreMesh(core_axis_name="c", subcore_axis_name="s")

    @functools.partial(
        pl.kernel,
        mesh=mesh,
        out_type=jax.ShapeDtypeStruct((batch, DIM), jnp.float32),
        scratch_types=[
            pltpu.VMEM((SUB, IPS), jnp.int32),
            pltpu.VMEM((C_ROWS, DIM), jnp.float32),
            pltpu.SemaphoreType.DMA,
        ],
    )
    def gather(x2_hbm, w_hbm, out_hbm, idx_v, rows_v, sem):
        wid = lax.axis_index("s") * NC + lax.axis_index("c")
        base_row = wid * bpw
        base_row2 = wid * rows2_pw

        def step(i, carry):
            pltpu.sync_copy(x2_hbm.at[pl.ds(base_row2 + i * SUB, SUB)], idx_v)
            copies = [
                pltpu.async_copy(
                    w_hbm.at[idx_v.at[j]],
                    rows_v.at[pl.ds(j * IPS, IPS)],
                    sem,
                )
                for j in range(SUB)
            ]
            for cp in copies:
                cp.wait()
            pltpu.sync_copy(
                rows_v, out_hbm.at[pl.ds(base_row + i * C_ROWS, C_ROWS)]
            )
            return carry

        lax.fori_loop(0, nsteps, step, 0)

    return gather


def kernel(x, weight):
    batch = x.shape[0] * x.shape[1]
    x2 = x.reshape(batch // IPS, IPS).astype(jnp.int32)
    out = _make_gather(batch)(x2, weight)
    return out.reshape(x.shape[0], x.shape[1], DIM)


# speedup vs baseline: 1.8458x; 1.8458x over previous
import functools

import jax
import jax.numpy as jnp
from jax import lax
from jax.experimental import pallas as pl
from jax.experimental.pallas import tpu as pltpu
from jax.experimental.pallas import tpu_sc as plsc

DIM = 64
NC = 2
NS = 16
NW = NC * NS
IPS = 128
SUB = 8
C_ROWS = SUB * IPS


def _make_gather(batch):
    bpw = batch // NW
    nsteps = bpw // C_ROWS
    rows2_pw = bpw // IPS

    mesh = plsc.VectorSubcoreMesh(core_axis_name="c", subcore_axis_name="s")

    @functools.partial(
        pl.kernel,
        mesh=mesh,
        out_type=jax.ShapeDtypeStruct((batch, DIM), jnp.float32),
        scratch_types=[
            pltpu.VMEM((SUB, IPS), jnp.int32),
            pltpu.VMEM((C_ROWS, DIM), jnp.float32),
            pltpu.SemaphoreType.DMA,
        ],
        compiler_params=pltpu.CompilerParams(use_tc_tiling_on_sc=False),
    )
    def gather(x2_hbm, w_hbm, out_hbm, idx_v, rows_v, sem):
        wid = lax.axis_index("s") * NC + lax.axis_index("c")
        base_row = wid * bpw
        base_row2 = wid * rows2_pw

        def step(i, carry):
            pltpu.sync_copy(x2_hbm.at[pl.ds(base_row2 + i * SUB, SUB)], idx_v)
            copies = [
                pltpu.async_copy(
                    w_hbm.at[idx_v.at[j]],
                    rows_v.at[pl.ds(j * IPS, IPS)],
                    sem,
                )
                for j in range(SUB)
            ]
            for cp in copies:
                cp.wait()
            pltpu.sync_copy(
                rows_v, out_hbm.at[pl.ds(base_row + i * C_ROWS, C_ROWS)]
            )
            return carry

        lax.fori_loop(0, nsteps, step, 0)

    return gather


def kernel(x, weight):
    batch = x.shape[0] * x.shape[1]
    x2 = x.reshape(batch // IPS, IPS).astype(jnp.int32)
    out = _make_gather(batch)(x2, weight)
    return out.reshape(x.shape[0], x.shape[1], DIM)


# speedup vs baseline: 1.8765x; 1.0166x over previous
import functools

import jax
import jax.numpy as jnp
from jax import lax
from jax.experimental import pallas as pl
from jax.experimental.pallas import tpu as pltpu
from jax.experimental.pallas import tpu_sc as plsc

DIM = 64
NC = 2
NS = 16
NW = NC * NS
IPS = 128
SUB = 4
C_ROWS = SUB * IPS


def _make_gather(batch):
    bpw = batch // NW
    npairs = bpw // (2 * C_ROWS)
    rows2_pw = bpw // IPS

    mesh = plsc.VectorSubcoreMesh(core_axis_name="c", subcore_axis_name="s")

    @functools.partial(
        pl.kernel,
        mesh=mesh,
        out_type=jax.ShapeDtypeStruct((batch, DIM), jnp.float32),
        scratch_types=[
            pltpu.VMEM((rows2_pw, IPS), jnp.int32),
            pltpu.VMEM((C_ROWS, DIM), jnp.float32),
            pltpu.VMEM((C_ROWS, DIM), jnp.float32),
            pltpu.SemaphoreType.DMA,
            pltpu.SemaphoreType.DMA,
            pltpu.SemaphoreType.DMA,
            pltpu.SemaphoreType.DMA,
        ],
        compiler_params=pltpu.CompilerParams(use_tc_tiling_on_sc=False),
    )
    def gather(x2_hbm, w_hbm, out_hbm, idx_v, buf0, buf1, g0, g1, s0, s1):
        wid = lax.axis_index("s") * NC + lax.axis_index("c")
        base_row = wid * bpw

        pltpu.sync_copy(x2_hbm.at[pl.ds(wid * rows2_pw, rows2_pw)], idx_v)

        def fire(k, buf, sem):
            return [
                pltpu.async_copy(
                    w_hbm.at[idx_v.at[k * SUB + j]],
                    buf.at[pl.ds(j * IPS, IPS)],
                    sem,
                )
                for j in range(SUB)
            ]

        def out_slot(k):
            return out_hbm.at[pl.ds(base_row + k * C_ROWS, C_ROWS)]

        def wait_bytes(buf, sem):
            pltpu.make_async_copy(buf, out_slot(0), sem).wait()

        def pair(p, carry):
            ka = 2 * p
            @pl.when(p > 0)
            def _():
                wait_bytes(buf0, s0)

            ga = fire(ka, buf0, g0)

            @pl.when(p > 0)
            def _():
                wait_bytes(buf1, g1)
                pltpu.async_copy(buf1, out_slot(ka - 1), s1)
                wait_bytes(buf1, s1)

            fire(ka + 1, buf1, g1)
            for cp in ga:
                cp.wait()
            pltpu.async_copy(buf0, out_slot(ka), s0)
            return carry

        lax.fori_loop(0, npairs, pair, 0)

        wait_bytes(buf0, s0)
        wait_bytes(buf1, g1)
        pltpu.async_copy(buf1, out_slot(2 * npairs - 1), s1)
        wait_bytes(buf1, s1)

    return gather


def kernel(x, weight):
    batch = x.shape[0] * x.shape[1]
    x2 = x.reshape(batch // IPS, IPS).astype(jnp.int32)
    out = _make_gather(batch)(x2, weight)
    return out.reshape(x.shape[0], x.shape[1], DIM)
